# baseline (speedup 1.0000x reference)
import functools

import jax
import jax.numpy as jnp
from jax import lax
from jax.experimental import pallas as pl
from jax.experimental.pallas import tpu as pltpu
from jax.experimental.pallas import tpu_sc as plsc

_WIN = 2736
_WPAD = 2816
_NWIN = 3


def _dist_argmin_body(x_ref, cbt_ref, xsq_ref, cbsq_ref,
                      idx_ref, losssum_ref, accv_s, acci_s, selv_s):
    i = pl.program_id(0)
    j = pl.program_id(1)
    bt = x_ref.shape[0]
    xb16 = x_ref[...].astype(jnp.bfloat16)
    cb16 = cbt_ref[...].astype(jnp.bfloat16)
    xc = lax.dot_general(xb16, cb16, (((1,), (0,)), ((), ())),
                         preferred_element_type=jnp.float32)
    dist = (cbsq_ref[0] + xsq_ref[...]) - 2.0 * xc
    m = jnp.min(dist, axis=1, keepdims=True)
    iota = lax.broadcasted_iota(jnp.int32, dist.shape, 1)
    li = jnp.min(jnp.where(dist == m, iota, 1 << 20), axis=1,
                 keepdims=True) + j * _WIN

    @pl.when(j == 0)
    def _():
        accv_s[...] = m.astype(jnp.bfloat16).astype(jnp.float32)
        acci_s[...] = li
        selv_s[...] = m

    @pl.when(j > 0)
    def _():
        av = accv_s[...]
        ai = acci_s[...]
        keep = (av < m) | ((av == m) & (ai < li))
        selv_s[...] = jnp.where(keep, selv_s[...], m)
        acci_s[...] = jnp.where(keep, ai, li)
        accv_s[...] = jnp.where(keep, av, m).astype(jnp.bfloat16).astype(jnp.float32)

    @pl.when(j == _NWIN - 1)
    def _():
        idx_ref[...] = acci_s[...].reshape(idx_ref.shape)

        @pl.when(i == 0)
        def _():
            losssum_ref[0, 0] = 0.0

        losssum_ref[0, 0] += jnp.sum(selv_s[...])


def _make_sc_gather(n, d, k):
    info = plsc.get_sparse_core_info()
    nc, ns, lanes = info.num_cores, info.num_subcores, info.num_lanes
    nw = nc * ns
    assert d % lanes == 0 and n % (8 * nw) == 0
    b_per_w = n // nw
    ch = min(128, b_per_w)
    n_ch = b_per_w // ch
    mesh = plsc.VectorSubcoreMesh(core_axis_name="c", subcore_axis_name="s")

    @functools.partial(
        pl.kernel, mesh=mesh,
        out_type=jax.ShapeDtypeStruct((n, d), jnp.float32),
        scratch_types=[
            pltpu.VMEM((ch,), jnp.int32),
            pltpu.VMEM((ch, d), jnp.float32),
            pltpu.SemaphoreType.DMA,
        ],
    )
    def gather(table_hbm, idx_hbm, out_hbm, idx_v, rows_v, sem):
        wid = lax.axis_index("s") * nc + lax.axis_index("c")
        base = wid * b_per_w
        for c in range(n_ch):
            off = base + c * ch
            pltpu.sync_copy(idx_hbm.at[pl.ds(off, ch)], idx_v)
            pltpu.async_copy(table_hbm.at[idx_v], rows_v, sem).wait()
            pltpu.sync_copy(rows_v, out_hbm.at[pl.ds(off, ch)])

    return gather


def kernel(x, codebook):
    d = x.shape[-1]
    k = codebook.shape[0]
    x2d = x.reshape(-1, d)
    n = x2d.shape[0]
    bt = 256
    grid = n // bt

    x_sqr = jnp.sum(x ** 2, axis=-1).reshape(n, 1)
    cb_sqr = jnp.sum(codebook ** 2, axis=1)

    cbt = codebook.T
    bounds = [(w * _WIN, min((w + 1) * _WIN, k)) for w in range(_NWIN)]
    cbt_pad = jnp.concatenate(
        [jnp.pad(cbt[:, lo:hi], ((0, 0), (0, _WPAD - (hi - lo)))) for lo, hi in bounds],
        axis=1)
    cbsq_pad = jnp.concatenate(
        [jnp.pad(cb_sqr[lo:hi], (0, _WPAD - (hi - lo)),
                 constant_values=jnp.inf) for lo, hi in bounds]
    ).reshape(_NWIN, 1, _WPAD)

    idx3, losssum = pl.pallas_call(
        _dist_argmin_body,
        grid=(grid, _NWIN),
        in_specs=[
            pl.BlockSpec((bt, d), lambda i, j: (i, 0)),
            pl.BlockSpec((d, _WPAD), lambda i, j: (0, j)),
            pl.BlockSpec((bt, 1), lambda i, j: (i, 0)),
            pl.BlockSpec((1, 1, _WPAD), lambda i, j: (j, 0, 0)),
        ],
        out_specs=[
            pl.BlockSpec((1, 1, bt), lambda i, j: (i, 0, 0)),
            pl.BlockSpec((1, 1), lambda i, j: (0, 0), memory_space=pltpu.SMEM),
        ],
        out_shape=[
            jax.ShapeDtypeStruct((grid, 1, bt), jnp.int32),
            jax.ShapeDtypeStruct((1, 1), jnp.float32),
        ],
        scratch_shapes=[
            pltpu.VMEM((bt, 1), jnp.float32),
            pltpu.VMEM((bt, 1), jnp.int32),
            pltpu.VMEM((bt, 1), jnp.float32),
        ],
    )(x2d, cbt_pad, x_sqr, cbsq_pad)

    indices = idx3.reshape(n)
    z_q = _make_sc_gather(n, d, k)(codebook, indices)
    loss = losssum[0, 0] / (n * d)
    return (z_q.reshape(x.shape), loss, loss, indices.reshape(x.shape[:-1]))


# speedup vs baseline: 1.2373x; 1.2373x over previous
import functools

import jax
import jax.numpy as jnp
from jax import lax
from jax.experimental import pallas as pl
from jax.experimental.pallas import tpu as pltpu
from jax.experimental.pallas import tpu_sc as plsc

_WIN = 2736
_WPAD = 2816
_NWIN = 3


def _dist_argmin_body(x_ref, cbt_ref, xsq_ref, cbsq_ref,
                      idx_ref, losssum_ref, accv_s, acci_s, selv_s):
    i = pl.program_id(0)
    j = pl.program_id(1)
    xb = x_ref[...]
    xb16 = (xb + xb).astype(jnp.bfloat16)
    cb16 = cbt_ref[...]
    xc2 = lax.dot_general(xb16, cb16, (((1,), (0,)), ((), ())),
                          preferred_element_type=jnp.float32)
    dist = (cbsq_ref[0] + xsq_ref[...]) - xc2
    m = jnp.min(dist, axis=1, keepdims=True)
    iota = lax.broadcasted_iota(jnp.int32, (1, dist.shape[1]), 1)
    li = jnp.min(jnp.where(dist == m, iota, 1 << 20), axis=1,
                 keepdims=True) + j * _WIN

    @pl.when(j == 0)
    def _():
        accv_s[...] = m.astype(jnp.bfloat16).astype(jnp.float32)
        acci_s[...] = li
        selv_s[...] = m

    @pl.when(j > 0)
    def _():
        av = accv_s[...]
        ai = acci_s[...]
        keep = (av < m) | ((av == m) & (ai < li))
        selv_s[...] = jnp.where(keep, selv_s[...], m)
        acci_s[...] = jnp.where(keep, ai, li)
        accv_s[...] = jnp.where(keep, av, m).astype(jnp.bfloat16).astype(jnp.float32)

    @pl.when(j == _NWIN - 1)
    def _():
        idx_ref[...] = acci_s[...].reshape(idx_ref.shape)

        @pl.when(i == 0)
        def _():
            losssum_ref[0, 0] = 0.0

        losssum_ref[0, 0] += jnp.sum(selv_s[...])


def _make_sc_gather(n, d, k):
    info = plsc.get_sparse_core_info()
    nc, ns, lanes = info.num_cores, info.num_subcores, info.num_lanes
    nw = nc * ns
    assert d % lanes == 0 and n % (8 * nw) == 0
    b_per_w = n // nw
    ch = min(128, b_per_w)
    n_ch = b_per_w // ch
    mesh = plsc.VectorSubcoreMesh(core_axis_name="c", subcore_axis_name="s")

    @functools.partial(
        pl.kernel, mesh=mesh,
        out_type=jax.ShapeDtypeStruct((n, d), jnp.float32),
        scratch_types=[
            pltpu.VMEM((ch,), jnp.int32),
            pltpu.VMEM((ch, d), jnp.float32),
            pltpu.SemaphoreType.DMA,
        ],
    )
    def gather(table_hbm, idx_hbm, out_hbm, idx_v, rows_v, sem):
        wid = lax.axis_index("s") * nc + lax.axis_index("c")
        base = wid * b_per_w
        for c in range(n_ch):
            off = base + c * ch
            pltpu.sync_copy(idx_hbm.at[pl.ds(off, ch)], idx_v)
            pltpu.async_copy(table_hbm.at[idx_v], rows_v, sem).wait()
            pltpu.sync_copy(rows_v, out_hbm.at[pl.ds(off, ch)])

    return gather


def kernel(x, codebook):
    d = x.shape[-1]
    k = codebook.shape[0]
    x2d = x.reshape(-1, d)
    n = x2d.shape[0]
    bt = 512
    grid = n // bt

    x_sqr = jnp.sum(x ** 2, axis=-1).reshape(n, 1)
    cb_sqr = jnp.sum(codebook ** 2, axis=1)

    cbt = codebook.T
    bounds = [(w * _WIN, min((w + 1) * _WIN, k)) for w in range(_NWIN)]
    cbt_pad = jnp.concatenate(
        [jnp.pad(cbt[:, lo:hi], ((0, 0), (0, _WPAD - (hi - lo)))) for lo, hi in bounds],
        axis=1).astype(jnp.bfloat16)
    cbsq_pad = jnp.concatenate(
        [jnp.pad(cb_sqr[lo:hi], (0, _WPAD - (hi - lo)),
                 constant_values=jnp.inf) for lo, hi in bounds]
    ).reshape(_NWIN, 1, _WPAD)

    idx3, losssum = pl.pallas_call(
        _dist_argmin_body,
        grid=(grid, _NWIN),
        in_specs=[
            pl.BlockSpec((bt, d), lambda i, j: (i, 0)),
            pl.BlockSpec((d, _WPAD), lambda i, j: (0, j)),
            pl.BlockSpec((bt, 1), lambda i, j: (i, 0)),
            pl.BlockSpec((1, 1, _WPAD), lambda i, j: (j, 0, 0)),
        ],
        out_specs=[
            pl.BlockSpec((1, 1, bt), lambda i, j: (i, 0, 0)),
            pl.BlockSpec((1, 1), lambda i, j: (0, 0), memory_space=pltpu.SMEM),
        ],
        out_shape=[
            jax.ShapeDtypeStruct((grid, 1, bt), jnp.int32),
            jax.ShapeDtypeStruct((1, 1), jnp.float32),
        ],
        scratch_shapes=[
            pltpu.VMEM((bt, 1), jnp.float32),
            pltpu.VMEM((bt, 1), jnp.int32),
            pltpu.VMEM((bt, 1), jnp.float32),
        ],
    )(x2d, cbt_pad, x_sqr, cbsq_pad)

    indices = idx3.reshape(n)
    z_q = _make_sc_gather(n, d, k)(codebook, indices)
    loss = losssum[0, 0] / (n * d)
    return (z_q.reshape(x.shape), loss, loss, indices.reshape(x.shape[:-1]))


# speedup vs baseline: 1.3492x; 1.0905x over previous
import functools

import jax
import jax.numpy as jnp
from jax import lax
from jax.experimental import pallas as pl
from jax.experimental.pallas import tpu as pltpu
from jax.experimental.pallas import tpu_sc as plsc

_WIN = 2736
_WPAD = 2816
_NWIN = 3


def _dist_argmin_body(x_ref, cbt_ref, xsq_ref, cbsq_ref,
                      idx_ref, losssum_ref, accv_s, acci_s, selv_s):
    i = pl.program_id(0)
    j = pl.program_id(1)
    xb = x_ref[...]
    xb16 = (xb + xb).astype(jnp.bfloat16)
    cb16 = cbt_ref[...]
    xc2 = lax.dot_general(xb16, cb16, (((1,), (0,)), ((), ())),
                          preferred_element_type=jnp.float32)
    dist = (cbsq_ref[0] + xsq_ref[...]) - xc2
    m = jnp.min(dist, axis=1, keepdims=True)
    bias = jnp.int32(0x3F800000)
    iota_k = lax.bitcast_convert_type(
        lax.broadcasted_iota(jnp.int32, (1, dist.shape[1]), 1) + bias,
        jnp.float32)
    big_k = lax.bitcast_convert_type(bias + jnp.int32(1 << 20), jnp.float32)
    key = jnp.min(jnp.where(dist == m, iota_k, big_k), axis=1,
                  keepdims=True)
    li = (lax.bitcast_convert_type(key, jnp.int32) - bias) + j * _WIN

    @pl.when(j == 0)
    def _():
        accv_s[...] = m.astype(jnp.bfloat16).astype(jnp.float32)
        acci_s[...] = li
        selv_s[...] = m

    @pl.when(j > 0)
    def _():
        av = accv_s[...]
        ai = acci_s[...]
        keep = (av < m) | ((av == m) & (ai < li))
        selv_s[...] = jnp.where(keep, selv_s[...], m)
        acci_s[...] = jnp.where(keep, ai, li)
        accv_s[...] = jnp.where(keep, av, m).astype(jnp.bfloat16).astype(jnp.float32)

    @pl.when(j == _NWIN - 1)
    def _():
        idx_ref[...] = acci_s[...].reshape(idx_ref.shape)

        @pl.when(i == 0)
        def _():
            losssum_ref[0, 0] = 0.0

        losssum_ref[0, 0] += jnp.sum(selv_s[...])


def _make_sc_gather(n, d, k):
    info = plsc.get_sparse_core_info()
    nc, ns, lanes = info.num_cores, info.num_subcores, info.num_lanes
    nw = nc * ns
    assert d % lanes == 0 and n % (8 * nw) == 0
    b_per_w = n // nw
    ch = min(128, b_per_w)
    n_ch = b_per_w // ch
    mesh = plsc.VectorSubcoreMesh(core_axis_name="c", subcore_axis_name="s")

    @functools.partial(
        pl.kernel, mesh=mesh,
        out_type=jax.ShapeDtypeStruct((n, d), jnp.float32),
        scratch_types=[
            pltpu.VMEM((ch,), jnp.int32),
            pltpu.VMEM((ch, d), jnp.float32),
            pltpu.SemaphoreType.DMA,
        ],
    )
    def gather(table_hbm, idx_hbm, out_hbm, idx_v, rows_v, sem):
        wid = lax.axis_index("s") * nc + lax.axis_index("c")
        base = wid * b_per_w
        for c in range(n_ch):
            off = base + c * ch
            pltpu.sync_copy(idx_hbm.at[pl.ds(off, ch)], idx_v)
            pltpu.async_copy(table_hbm.at[idx_v], rows_v, sem).wait()
            pltpu.sync_copy(rows_v, out_hbm.at[pl.ds(off, ch)])

    return gather


def kernel(x, codebook):
    d = x.shape[-1]
    k = codebook.shape[0]
    x2d = x.reshape(-1, d)
    n = x2d.shape[0]
    bt = 512
    grid = n // bt

    x_sqr = jnp.sum(x ** 2, axis=-1).reshape(n, 1)
    cb_sqr = jnp.sum(codebook ** 2, axis=1)

    cbt = codebook.T
    bounds = [(w * _WIN, min((w + 1) * _WIN, k)) for w in range(_NWIN)]
    cbt_pad = jnp.concatenate(
        [jnp.pad(cbt[:, lo:hi], ((0, 0), (0, _WPAD - (hi - lo)))) for lo, hi in bounds],
        axis=1).astype(jnp.bfloat16)
    cbsq_pad = jnp.concatenate(
        [jnp.pad(cb_sqr[lo:hi], (0, _WPAD - (hi - lo)),
                 constant_values=jnp.inf) for lo, hi in bounds]
    ).reshape(_NWIN, 1, _WPAD)

    idx3, losssum = pl.pallas_call(
        _dist_argmin_body,
        grid=(grid, _NWIN),
        in_specs=[
            pl.BlockSpec((bt, d), lambda i, j: (i, 0)),
            pl.BlockSpec((d, _WPAD), lambda i, j: (0, j)),
            pl.BlockSpec((bt, 1), lambda i, j: (i, 0)),
            pl.BlockSpec((1, 1, _WPAD), lambda i, j: (j, 0, 0)),
        ],
        out_specs=[
            pl.BlockSpec((1, 1, bt), lambda i, j: (i, 0, 0)),
            pl.BlockSpec((1, 1), lambda i, j: (0, 0), memory_space=pltpu.SMEM),
        ],
        out_shape=[
            jax.ShapeDtypeStruct((grid, 1, bt), jnp.int32),
            jax.ShapeDtypeStruct((1, 1), jnp.float32),
        ],
        scratch_shapes=[
            pltpu.VMEM((bt, 1), jnp.float32),
            pltpu.VMEM((bt, 1), jnp.int32),
            pltpu.VMEM((bt, 1), jnp.float32),
        ],
    )(x2d, cbt_pad, x_sqr, cbsq_pad)

    indices = idx3.reshape(n)
    z_q = _make_sc_gather(n, d, k)(codebook, indices)
    loss = losssum[0, 0] / (n * d)
    return (z_q.reshape(x.shape), loss, loss, indices.reshape(x.shape[:-1]))


# speedup vs baseline: 1.3586x; 1.0069x over previous
import functools

import jax
import jax.numpy as jnp
from jax import lax
from jax.experimental import pallas as pl
from jax.experimental.pallas import tpu as pltpu
from jax.experimental.pallas import tpu_sc as plsc

_WIN = 2736
_WPAD = 2816
_NWIN = 3


def _dist_argmin_body(x_ref, cbt_ref, xsq_ref, cbsq_ref,
                      idx_ref, losssum_ref, accv_s, acci_s, selv_s):
    i = pl.program_id(0)
    j = pl.program_id(1)
    xb = x_ref[...]
    xb16 = (xb + xb).astype(jnp.bfloat16)
    cb16 = cbt_ref[...]
    xc2 = lax.dot_general(xb16, cb16, (((1,), (0,)), ((), ())),
                          preferred_element_type=jnp.float32)
    dist = (cbsq_ref[0] + xsq_ref[...]) - xc2
    m = jnp.min(dist, axis=1, keepdims=True)
    bias = jnp.int32(0x3F800000)
    iota_k = lax.bitcast_convert_type(
        lax.broadcasted_iota(jnp.int32, (1, dist.shape[1]), 1) + bias,
        jnp.float32)
    big_k = lax.bitcast_convert_type(bias + jnp.int32(1 << 20), jnp.float32)
    key = jnp.min(jnp.where(dist == m, iota_k, big_k), axis=1,
                  keepdims=True)
    li = (lax.bitcast_convert_type(key, jnp.int32) - bias) + j * _WIN

    @pl.when(j == 0)
    def _():
        accv_s[...] = m.astype(jnp.bfloat16).astype(jnp.float32)
        acci_s[...] = li
        selv_s[...] = m

    @pl.when(j > 0)
    def _():
        av = accv_s[...]
        ai = acci_s[...]
        keep = (av < m) | ((av == m) & (ai < li))
        selv_s[...] = jnp.where(keep, selv_s[...], m)
        acci_s[...] = jnp.where(keep, ai, li)
        accv_s[...] = jnp.where(keep, av, m).astype(jnp.bfloat16).astype(jnp.float32)

    @pl.when(j == _NWIN - 1)
    def _():
        idx_ref[...] = acci_s[...].reshape(idx_ref.shape)

        @pl.when(i == 0)
        def _():
            losssum_ref[0, 0] = 0.0

        losssum_ref[0, 0] += jnp.sum(selv_s[...])


def _make_sc_gather(n, d, k):
    info = plsc.get_sparse_core_info()
    nc, ns, lanes = info.num_cores, info.num_subcores, info.num_lanes
    nw = nc * ns
    assert d % lanes == 0 and n % (8 * nw) == 0
    b_per_w = n // nw
    ch = min(128, b_per_w)
    n_ch = b_per_w // ch
    mesh = plsc.VectorSubcoreMesh(core_axis_name="c", subcore_axis_name="s")

    @functools.partial(
        pl.kernel, mesh=mesh,
        out_type=jax.ShapeDtypeStruct((n, d), jnp.float32),
        scratch_types=[
            pltpu.VMEM((n_ch, ch), jnp.int32),
            pltpu.VMEM((ch, d), jnp.float32),
            pltpu.VMEM((ch, d), jnp.float32),
            pltpu.SemaphoreType.DMA,
            pltpu.SemaphoreType.DMA,
            pltpu.SemaphoreType.DMA,
            pltpu.SemaphoreType.DMA,
        ],
    )
    def gather(table_hbm, idx_hbm, out_hbm, idx_v, buf0, buf1,
               sg0, sg1, ss0, ss1):
        wid = lax.axis_index("s") * nc + lax.axis_index("c")
        base = wid * b_per_w
        bufs, gsems, ssems = [buf0, buf1], [sg0, sg1], [ss0, ss1]
        pltpu.sync_copy(idx_hbm.at[wid], idx_v)
        gcop = [None, None]
        scop = [None, None]
        gcop[0] = pltpu.async_copy(table_hbm.at[idx_v.at[0]], buf0, sg0)
        for c in range(n_ch):
            b = c & 1
            nb = (c + 1) & 1
            if c + 1 < n_ch:
                if scop[nb] is not None:
                    scop[nb].wait()
                gcop[nb] = pltpu.async_copy(
                    table_hbm.at[idx_v.at[c + 1]], bufs[nb], gsems[nb])
            gcop[b].wait()
            scop[b] = pltpu.async_copy(
                bufs[b], out_hbm.at[pl.ds(base + c * ch, ch)], ssems[b])
        for s in scop:
            if s is not None:
                s.wait()

    return gather


def kernel(x, codebook):
    d = x.shape[-1]
    k = codebook.shape[0]
    x2d = x.reshape(-1, d)
    n = x2d.shape[0]
    bt = 512
    grid = n // bt

    x_sqr = jnp.sum(x ** 2, axis=-1).reshape(n, 1)
    cb_sqr = jnp.sum(codebook ** 2, axis=1)

    cbt = codebook.T
    bounds = [(w * _WIN, min((w + 1) * _WIN, k)) for w in range(_NWIN)]
    cbt_pad = jnp.concatenate(
        [jnp.pad(cbt[:, lo:hi], ((0, 0), (0, _WPAD - (hi - lo)))) for lo, hi in bounds],
        axis=1).astype(jnp.bfloat16)
    cbsq_pad = jnp.concatenate(
        [jnp.pad(cb_sqr[lo:hi], (0, _WPAD - (hi - lo)),
                 constant_values=jnp.inf) for lo, hi in bounds]
    ).reshape(_NWIN, 1, _WPAD)

    idx3, losssum = pl.pallas_call(
        _dist_argmin_body,
        grid=(grid, _NWIN),
        in_specs=[
            pl.BlockSpec((bt, d), lambda i, j: (i, 0)),
            pl.BlockSpec((d, _WPAD), lambda i, j: (0, j)),
            pl.BlockSpec((bt, 1), lambda i, j: (i, 0)),
            pl.BlockSpec((1, 1, _WPAD), lambda i, j: (j, 0, 0)),
        ],
        out_specs=[
            pl.BlockSpec((1, 1, bt), lambda i, j: (i, 0, 0)),
            pl.BlockSpec((1, 1), lambda i, j: (0, 0), memory_space=pltpu.SMEM),
        ],
        out_shape=[
            jax.ShapeDtypeStruct((grid, 1, bt), jnp.int32),
            jax.ShapeDtypeStruct((1, 1), jnp.float32),
        ],
        scratch_shapes=[
            pltpu.VMEM((bt, 1), jnp.float32),
            pltpu.VMEM((bt, 1), jnp.int32),
            pltpu.VMEM((bt, 1), jnp.float32),
        ],
    )(x2d, cbt_pad, x_sqr, cbsq_pad)

    indices = idx3.reshape(n)
    nw = 32
    z_q = _make_sc_gather(n, d, k)(
        codebook, indices.reshape(nw, n // (nw * 128), 128))
    loss = losssum[0, 0] / (n * d)
    return (z_q.reshape(x.shape), loss, loss, indices.reshape(x.shape[:-1]))


# speedup vs baseline: 1.4256x; 1.0493x over previous
import functools

import jax
import jax.numpy as jnp
from jax import lax
from jax.experimental import pallas as pl
from jax.experimental.pallas import tpu as pltpu
from jax.experimental.pallas import tpu_sc as plsc

_WIN = 2736
_WPAD = 2816
_NWIN = 3


def _dist_argmin_body(x_ref, cbt_ref, xsq_ref, cbsq_ref,
                      idx_ref, losssum_ref, accv_s, acci_s, selv_s):
    i = pl.program_id(0)
    j = pl.program_id(1)
    xb = x_ref[...]
    xb16 = (xb + xb).astype(jnp.bfloat16)
    cb16 = cbt_ref[...]
    xc2 = lax.dot_general(xb16, cb16, (((1,), (0,)), ((), ())),
                          preferred_element_type=jnp.float32)
    dist = (cbsq_ref[0] + xsq_ref[...]) - xc2
    m = jnp.min(dist, axis=1, keepdims=True)
    bias = jnp.int32(0x3F800000)
    iota_k = lax.bitcast_convert_type(
        lax.broadcasted_iota(jnp.int32, (1, dist.shape[1]), 1) + bias,
        jnp.float32)
    big_k = lax.bitcast_convert_type(bias + jnp.int32(1 << 20), jnp.float32)
    key = jnp.min(jnp.where(dist == m, iota_k, big_k), axis=1,
                  keepdims=True)
    li = (lax.bitcast_convert_type(key, jnp.int32) - bias) + j * _WIN

    @pl.when(j == 0)
    def _():
        accv_s[...] = m.astype(jnp.bfloat16).astype(jnp.float32)
        acci_s[...] = li
        selv_s[...] = m

    @pl.when(j > 0)
    def _():
        av = accv_s[...]
        ai = acci_s[...]
        keep = (av < m) | ((av == m) & (ai < li))
        selv_s[...] = jnp.where(keep, selv_s[...], m)
        acci_s[...] = jnp.where(keep, ai, li)
        accv_s[...] = jnp.where(keep, av, m).astype(jnp.bfloat16).astype(jnp.float32)

    @pl.when(j == _NWIN - 1)
    def _():
        idx_ref[...] = acci_s[...].reshape(idx_ref.shape)

        @pl.when(i == 0)
        def _():
            losssum_ref[0, 0] = 0.0

        losssum_ref[0, 0] += jnp.sum(selv_s[...])


def _make_sc_gather(n, d, k):
    info = plsc.get_sparse_core_info()
    nc, ns, lanes = info.num_cores, info.num_subcores, info.num_lanes
    nw = nc * ns
    assert d % lanes == 0 and n % (8 * nw) == 0
    b_per_w = n // nw
    ch = min(128, b_per_w)
    n_ch = b_per_w // ch
    mesh = plsc.VectorSubcoreMesh(core_axis_name="c", subcore_axis_name="s")

    @functools.partial(
        pl.kernel, mesh=mesh,
        out_type=jax.ShapeDtypeStruct((n, d), jnp.float32),
        scratch_types=[
            pltpu.VMEM((n_ch, ch), jnp.int32),
            pltpu.VMEM((ch, d), jnp.float32),
            pltpu.VMEM((ch, d), jnp.float32),
            pltpu.SemaphoreType.DMA,
            pltpu.SemaphoreType.DMA,
            pltpu.SemaphoreType.DMA,
            pltpu.SemaphoreType.DMA,
        ],
    )
    def gather(table_hbm, idx_hbm, out_hbm, idx_v, buf0, buf1,
               sg0, sg1, ss0, ss1):
        wid = lax.axis_index("s") * nc + lax.axis_index("c")
        base = wid * b_per_w
        bufs, gsems, ssems = [buf0, buf1], [sg0, sg1], [ss0, ss1]
        pltpu.sync_copy(idx_hbm.at[wid], idx_v)
        gcop = [None, None]
        scop = [None, None]
        gcop[0] = pltpu.async_copy(table_hbm.at[idx_v.at[0]], buf0, sg0)
        for c in range(n_ch):
            b = c & 1
            nb = (c + 1) & 1
            if c + 1 < n_ch:
                if scop[nb] is not None:
                    scop[nb].wait()
                gcop[nb] = pltpu.async_copy(
                    table_hbm.at[idx_v.at[c + 1]], bufs[nb], gsems[nb])
            gcop[b].wait()
            scop[b] = pltpu.async_copy(
                bufs[b], out_hbm.at[pl.ds(base + c * ch, ch)], ssems[b])
        for s in scop:
            if s is not None:
                s.wait()

    return gather


def kernel(x, codebook):
    d = x.shape[-1]
    k = codebook.shape[0]
    x2d = x.reshape(-1, d)
    n = x2d.shape[0]
    bt = 1024
    grid = n // bt

    x_sqr = jnp.sum(x ** 2, axis=-1).reshape(n, 1)
    cb_sqr = jnp.sum(codebook ** 2, axis=1)

    cbt = codebook.T
    bounds = [(w * _WIN, min((w + 1) * _WIN, k)) for w in range(_NWIN)]
    cbt_pad = jnp.concatenate(
        [jnp.pad(cbt[:, lo:hi], ((0, 0), (0, _WPAD - (hi - lo)))) for lo, hi in bounds],
        axis=1).astype(jnp.bfloat16)
    cbsq_pad = jnp.concatenate(
        [jnp.pad(cb_sqr[lo:hi], (0, _WPAD - (hi - lo)),
                 constant_values=jnp.inf) for lo, hi in bounds]
    ).reshape(_NWIN, 1, _WPAD)

    idx3, losssum = pl.pallas_call(
        _dist_argmin_body,
        grid=(grid, _NWIN),
        in_specs=[
            pl.BlockSpec((bt, d), lambda i, j: (i, 0)),
            pl.BlockSpec((d, _WPAD), lambda i, j: (0, j)),
            pl.BlockSpec((bt, 1), lambda i, j: (i, 0)),
            pl.BlockSpec((1, 1, _WPAD), lambda i, j: (j, 0, 0)),
        ],
        out_specs=[
            pl.BlockSpec((1, 1, bt), lambda i, j: (i, 0, 0)),
            pl.BlockSpec((1, 1), lambda i, j: (0, 0), memory_space=pltpu.SMEM),
        ],
        out_shape=[
            jax.ShapeDtypeStruct((grid, 1, bt), jnp.int32),
            jax.ShapeDtypeStruct((1, 1), jnp.float32),
        ],
        scratch_shapes=[
            pltpu.VMEM((bt, 1), jnp.float32),
            pltpu.VMEM((bt, 1), jnp.int32),
            pltpu.VMEM((bt, 1), jnp.float32),
        ],
    )(x2d, cbt_pad, x_sqr, cbsq_pad)

    indices = idx3.reshape(n)
    nw = 32
    z_q = _make_sc_gather(n, d, k)(
        codebook, indices.reshape(nw, n // (nw * 128), 128))
    loss = losssum[0, 0] / (n * d)
    return (z_q.reshape(x.shape), loss, loss, indices.reshape(x.shape[:-1]))


# speedup vs baseline: 1.4915x; 1.0462x over previous
import functools

import jax
import jax.numpy as jnp
from jax import lax
from jax.experimental import pallas as pl
from jax.experimental.pallas import tpu as pltpu
from jax.experimental.pallas import tpu_sc as plsc

_WIN = 2736
_WPAD = 2816
_NWIN = 3


def _dist_argmin_body(x_ref, cbt_ref, xsq_ref, cbsq_ref,
                      idx_ref, losssum_ref, accv_s, acci_s, selv_s):
    i = pl.program_id(0)
    j = pl.program_id(1)
    xb = x_ref[...]
    xb16 = (xb + xb).astype(jnp.bfloat16)
    cb16 = cbt_ref[...]
    xc2 = lax.dot_general(xb16, cb16, (((1,), (0,)), ((), ())),
                          preferred_element_type=jnp.float32)
    dist = (cbsq_ref[0] + xsq_ref[...]) - xc2
    m = jnp.min(dist, axis=1, keepdims=True)
    bias = jnp.int32(0x3F800000)
    iota_k = lax.bitcast_convert_type(
        lax.broadcasted_iota(jnp.int32, (1, dist.shape[1]), 1) + bias,
        jnp.float32)
    big_k = lax.bitcast_convert_type(bias + jnp.int32(1 << 20), jnp.float32)
    key = jnp.min(jnp.where(dist == m, iota_k, big_k), axis=1,
                  keepdims=True)
    li = (lax.bitcast_convert_type(key, jnp.int32) - bias) + j * _WIN

    @pl.when(j == 0)
    def _():
        accv_s[...] = m.astype(jnp.bfloat16).astype(jnp.float32)
        acci_s[...] = li
        selv_s[...] = m

    @pl.when(j > 0)
    def _():
        av = accv_s[...]
        ai = acci_s[...]
        keep = (av < m) | ((av == m) & (ai < li))
        selv_s[...] = jnp.where(keep, selv_s[...], m)
        acci_s[...] = jnp.where(keep, ai, li)
        accv_s[...] = jnp.where(keep, av, m).astype(jnp.bfloat16).astype(jnp.float32)

    @pl.when(j == _NWIN - 1)
    def _():
        idx_ref[...] = acci_s[...].reshape(idx_ref.shape)

        @pl.when(i == 0)
        def _():
            losssum_ref[0, 0] = 0.0

        losssum_ref[0, 0] += jnp.sum(selv_s[...])


def _make_sc_gather(n, d, k):
    info = plsc.get_sparse_core_info()
    nc, ns, lanes = info.num_cores, info.num_subcores, info.num_lanes
    nw = nc * ns
    assert d % lanes == 0 and n % (8 * nw) == 0
    b_per_w = n // nw
    ch = min(128, b_per_w)
    n_ch = b_per_w // ch
    mesh = plsc.VectorSubcoreMesh(core_axis_name="c", subcore_axis_name="s")

    @functools.partial(
        pl.kernel, mesh=mesh,
        out_type=jax.ShapeDtypeStruct((n, d), jnp.float32),
        scratch_types=[
            pltpu.VMEM((n_ch, ch), jnp.int32),
            pltpu.VMEM((ch, d), jnp.float32),
            pltpu.VMEM((ch, d), jnp.float32),
            pltpu.SemaphoreType.DMA,
            pltpu.SemaphoreType.DMA,
            pltpu.SemaphoreType.DMA,
            pltpu.SemaphoreType.DMA,
        ],
    )
    def gather(table_hbm, idx_hbm, out_hbm, idx_v, buf0, buf1,
               sg0, sg1, ss0, ss1):
        wid = lax.axis_index("s") * nc + lax.axis_index("c")
        base = wid * b_per_w
        bufs, gsems, ssems = [buf0, buf1], [sg0, sg1], [ss0, ss1]
        pltpu.sync_copy(idx_hbm.at[wid], idx_v)
        gcop = [None, None]
        scop = [None, None]
        gcop[0] = pltpu.async_copy(table_hbm.at[idx_v.at[0]], buf0, sg0)
        for c in range(n_ch):
            b = c & 1
            nb = (c + 1) & 1
            if c + 1 < n_ch:
                if scop[nb] is not None:
                    scop[nb].wait()
                gcop[nb] = pltpu.async_copy(
                    table_hbm.at[idx_v.at[c + 1]], bufs[nb], gsems[nb])
            gcop[b].wait()
            scop[b] = pltpu.async_copy(
                bufs[b], out_hbm.at[pl.ds(base + c * ch, ch)], ssems[b])
        for s in scop:
            if s is not None:
                s.wait()

    return gather


def kernel(x, codebook):
    d = x.shape[-1]
    k = codebook.shape[0]
    x2d = x.reshape(-1, d)
    n = x2d.shape[0]
    bt = 2048
    grid = n // bt

    x_sqr = jnp.sum(x ** 2, axis=-1).reshape(n, 1)
    cb_sqr = jnp.sum(codebook ** 2, axis=1)

    cbt = codebook.T
    bounds = [(w * _WIN, min((w + 1) * _WIN, k)) for w in range(_NWIN)]
    cbt_pad = jnp.concatenate(
        [jnp.pad(cbt[:, lo:hi], ((0, 0), (0, _WPAD - (hi - lo)))) for lo, hi in bounds],
        axis=1).astype(jnp.bfloat16)
    cbsq_pad = jnp.concatenate(
        [jnp.pad(cb_sqr[lo:hi], (0, _WPAD - (hi - lo)),
                 constant_values=jnp.inf) for lo, hi in bounds]
    ).reshape(_NWIN, 1, _WPAD)

    idx3, losssum = pl.pallas_call(
        _dist_argmin_body,
        grid=(grid, _NWIN),
        in_specs=[
            pl.BlockSpec((bt, d), lambda i, j: (i, 0)),
            pl.BlockSpec((d, _WPAD), lambda i, j: (0, j)),
            pl.BlockSpec((bt, 1), lambda i, j: (i, 0)),
            pl.BlockSpec((1, 1, _WPAD), lambda i, j: (j, 0, 0)),
        ],
        out_specs=[
            pl.BlockSpec((1, 1, bt), lambda i, j: (i, 0, 0)),
            pl.BlockSpec((1, 1), lambda i, j: (0, 0), memory_space=pltpu.SMEM),
        ],
        out_shape=[
            jax.ShapeDtypeStruct((grid, 1, bt), jnp.int32),
            jax.ShapeDtypeStruct((1, 1), jnp.float32),
        ],
        scratch_shapes=[
            pltpu.VMEM((bt, 1), jnp.float32),
            pltpu.VMEM((bt, 1), jnp.int32),
            pltpu.VMEM((bt, 1), jnp.float32),
        ],
    )(x2d, cbt_pad, x_sqr, cbsq_pad)

    indices = idx3.reshape(n)
    nw = 32
    z_q = _make_sc_gather(n, d, k)(
        codebook, indices.reshape(nw, n // (nw * 128), 128))
    loss = losssum[0, 0] / (n * d)
    return (z_q.reshape(x.shape), loss, loss, indices.reshape(x.shape[:-1]))
